# baseline (speedup 1.0000x reference)
import functools

import jax
import jax.numpy as jnp
from jax import lax
from jax.experimental import pallas as pl
from jax.experimental.pallas import tpu as pltpu
from jax.experimental.pallas import tpu_sc as plsc

_ALPHA = 4
_PARTS = 16
_GROUP = 8


_NBUF = 3


def _slow_gather(frames4, src_rows, n_chunks, d):
    info = plsc.get_sparse_core_info()
    nc, ns = info.num_cores, info.num_subcores
    nw = nc * ns
    cpw = n_chunks // nw
    n_groups = cpw // _GROUP
    mesh = plsc.VectorSubcoreMesh(core_axis_name="c", subcore_axis_name="s")

    @functools.partial(
        pl.kernel,
        mesh=mesh,
        out_type=jax.ShapeDtypeStruct((n_chunks, d), jnp.float32),
        scratch_types=[
            pltpu.VMEM((cpw,), jnp.int32),
            [pltpu.VMEM((_GROUP, d), jnp.float32) for _ in range(_NBUF)],
            [pltpu.SemaphoreType.DMA for _ in range(_NBUF)],
            [pltpu.SemaphoreType.DMA for _ in range(_NBUF)],
        ],
    )
    def body(src_hbm, frames_hbm, out_hbm, idx_v, bufs, gsems, wsems):
        wid = lax.axis_index("s") * nc + lax.axis_index("c")
        base = wid * cpw
        pltpu.sync_copy(src_hbm.at[pl.ds(base, cpw)], idx_v)
        gathers = [None] * n_groups
        writes = [None] * n_groups
        for g in range(min(_NBUF, n_groups)):
            gathers[g] = pltpu.async_copy(
                frames_hbm.at[idx_v.at[pl.ds(g * _GROUP, _GROUP)]],
                bufs[g % _NBUF], gsems[g % _NBUF])
        for g in range(n_groups):
            gathers[g].wait()
            writes[g] = pltpu.async_copy(
                bufs[g % _NBUF], out_hbm.at[pl.ds(base + g * _GROUP, _GROUP)],
                wsems[g % _NBUF])
            nxt = g + _NBUF
            if nxt < n_groups:
                writes[g].wait()
                gathers[nxt] = pltpu.async_copy(
                    frames_hbm.at[idx_v.at[pl.ds(nxt * _GROUP, _GROUP)]],
                    bufs[nxt % _NBUF], gsems[nxt % _NBUF])
        for g in range(max(0, n_groups - _NBUF), n_groups):
            writes[g].wait()

    return body(src_rows, frames4)


def kernel(frames):
    ch, t, h, w = frames.shape
    n_sel = t // _ALPHA
    idx = jnp.linspace(0, t - 1, n_sel).astype(jnp.int32)

    d = (h * w) // _PARTS
    n_chunks = ch * n_sel * _PARTS
    o = jnp.arange(n_chunks, dtype=jnp.int32)
    c = o // (n_sel * _PARTS)
    s = (o // _PARTS) % n_sel
    p = o % _PARTS
    src_rows = (c * t + idx[s]) * _PARTS + p

    frames4 = frames.reshape(ch * t * _PARTS, d)
    slow4 = _slow_gather(frames4, src_rows, n_chunks, d)
    slow = slow4.reshape(ch, n_sel, h, w)
    return (slow, frames)


# speedup vs baseline: 2.2658x; 2.2658x over previous
import functools

import jax
import jax.numpy as jnp
from jax import lax
from jax.experimental import pallas as pl
from jax.experimental.pallas import tpu as pltpu
from jax.experimental.pallas import tpu_sc as plsc

_ALPHA = 4
_HALves = 2
_NBUF = 3


def _pack_pathway_sc(frames3, n_rows, t_len, n_sel, h, w):
    info = plsc.get_sparse_core_info()
    nc, ns = info.num_cores, info.num_subcores
    nw = nc * ns
    rpw = n_rows // nw
    nch = rpw * _HALves
    hh = h // _HALves
    mesh = plsc.VectorSubcoreMesh(core_axis_name="c", subcore_axis_name="s")

    @functools.partial(
        pl.kernel,
        mesh=mesh,
        out_type=(
            jax.ShapeDtypeStruct((n_rows, h, w), jnp.float32),
            jax.ShapeDtypeStruct((n_rows // _ALPHA, h, w), jnp.float32)
        ),
        scratch_types=[
            [pltpu.VMEM((hh, w), jnp.float32) for _ in range(_NBUF)],
            [pltpu.SemaphoreType.DMA for _ in range(_NBUF)],
        ],
    )
    def body(frames_hbm, fast_hbm, slow_hbm, bufs, rsems):
        wid = lax.axis_index("s") * nc + lax.axis_index("c")
        base = wid * rpw
        reads = [None] * nch

        def start_read(i):
            r = base + i // _HALves
            hp = (i % _HALves) * hh
            return pltpu.async_copy(
                frames_hbm.at[r, pl.ds(hp, hh), :], bufs[i % _NBUF],
                rsems[i % _NBUF])

        for i in range(_NBUF):
            reads[i] = start_read(i)
        for i in range(nch):
            r = base + i // _HALves
            hp = (i % _HALves) * hh
            t = r % t_len
            c = r // t_len
            s_c = (t * (n_sel - 1) + (t_len - 2)) // (t_len - 1)
            sel = ((s_c * (t_len - 1)) // (n_sel - 1)) == t
            reads[i].wait()
            pltpu.sync_copy(bufs[i % _NBUF], fast_hbm.at[r, pl.ds(hp, hh), :])

            @pl.when(sel)
            def _():
                pltpu.sync_copy(bufs[i % _NBUF],
                                slow_hbm.at[c * n_sel + s_c, pl.ds(hp, hh), :])

            if i + _NBUF < nch:
                reads[i + _NBUF] = start_read(i + _NBUF)

    return body(frames3)


def kernel(frames):
    ch, t_len, h, w = frames.shape
    n_sel = t_len // _ALPHA
    frames3 = frames.reshape(ch * t_len, h, w)
    fast3, slow3 = _pack_pathway_sc(frames3, ch * t_len, t_len, n_sel, h, w)
    return (slow3.reshape(ch, n_sel, h, w), fast3.reshape(ch, t_len, h, w))


# speedup vs baseline: 2.2665x; 1.0003x over previous
import functools

import jax
import jax.numpy as jnp
from jax import lax
from jax.experimental import pallas as pl
from jax.experimental.pallas import tpu as pltpu
from jax.experimental.pallas import tpu_sc as plsc

_ALPHA = 4
_HPARTS = 2
_NBUF = 3


def _fast_copy_sc(frames3, n_rows, h, w):
    info = plsc.get_sparse_core_info()
    nc, ns = info.num_cores, info.num_subcores
    nw = nc * ns
    rpw = n_rows // nw
    nch = rpw * _HPARTS
    hh = h // _HPARTS
    mesh = plsc.VectorSubcoreMesh(core_axis_name="c", subcore_axis_name="s")

    @functools.partial(
        pl.kernel,
        mesh=mesh,
        out_type=jax.ShapeDtypeStruct((n_rows, h, w), jnp.float32),
        scratch_types=[
            [pltpu.VMEM((hh, w), jnp.float32) for _ in range(_NBUF)],
            [pltpu.SemaphoreType.DMA for _ in range(_NBUF)],
        ],
    )
    def body(frames_hbm, fast_hbm, bufs, rsems):
        wid = lax.axis_index("s") * nc + lax.axis_index("c")
        base = wid * rpw
        reads = [None] * nch

        def start_read(i):
            r = base + i // _HPARTS
            hp = (i % _HPARTS) * hh
            return pltpu.async_copy(
                frames_hbm.at[r, pl.ds(hp, hh), :], bufs[i % _NBUF],
                rsems[i % _NBUF])

        for i in range(_NBUF):
            reads[i] = start_read(i)
        for i in range(nch):
            r = base + i // _HPARTS
            hp = (i % _HPARTS) * hh
            reads[i].wait()
            pltpu.sync_copy(bufs[i % _NBUF], fast_hbm.at[r, pl.ds(hp, hh), :])
            if i + _NBUF < nch:
                reads[i + _NBUF] = start_read(i + _NBUF)

    return body(frames3)


def _slow_gather_tc(frames, ch, t_len, n_sel, h, w):
    def body(in_ref, out_ref):
        out_ref[...] = in_ref[...]

    return pl.pallas_call(
        body,
        grid=(n_sel,),
        in_specs=[pl.BlockSpec(
            (ch, 1, h, w),
            lambda s: (0, (s * (t_len - 1)) // (n_sel - 1), 0, 0))],
        out_specs=pl.BlockSpec((ch, 1, h, w), lambda s: (0, s, 0, 0)),
        out_shape=jax.ShapeDtypeStruct((ch, n_sel, h, w), jnp.float32),
    )(frames)


def kernel(frames):
    ch, t_len, h, w = frames.shape
    n_sel = t_len // _ALPHA
    frames3 = frames.reshape(ch * t_len, h, w)
    fast3 = _fast_copy_sc(frames3, ch * t_len, h, w)
    slow = _slow_gather_tc(frames, ch, t_len, n_sel, h, w)
    return (slow, fast3.reshape(ch, t_len, h, w))
